# baseline (speedup 1.0000x reference)
import functools

import jax
import jax.numpy as jnp
from jax import lax
from jax.experimental import pallas as pl
from jax.experimental.pallas import tpu as pltpu
from jax.experimental.pallas import tpu_sc as plsc

_VOCAB = 32128
_DIM = 4096
_B = 4
_S = 512
_N = _B * _S
_NC = 2
_NS = 16
_NW = _NC * _NS
_RPW = _N // _NW
_WPB = _S // _RPW
_R = 2
_NCHUNK = _RPW // _R
_NBUF = 12

_mesh = plsc.VectorSubcoreMesh(core_axis_name="c", subcore_axis_name="s")


@functools.partial(
    pl.kernel,
    out_type=jax.ShapeDtypeStruct((_B, _S, _DIM), jnp.float32),
    mesh=_mesh,
    scratch_types=[
        pltpu.VMEM((_NCHUNK, _R), jnp.int32),
        pltpu.VMEM((_NBUF, _R, _DIM), jnp.float32),
        [pltpu.SemaphoreType.DMA] * _NBUF,
        [pltpu.SemaphoreType.DMA] * _NBUF,
    ],
)
def _emb_lookup(table_hbm, idx_hbm, out_hbm, idx_v, rows_v, gsems, osems):
    wid = lax.axis_index("s") * _NC + lax.axis_index("c")
    b = wid // _WPB
    s0 = (wid % _WPB) * _RPW
    pltpu.sync_copy(idx_hbm.at[pl.ds(wid * _NCHUNK, _NCHUNK)], idx_v)

    def gather(c):
        return pltpu.async_copy(
            table_hbm.at[idx_v.at[c]],
            rows_v.at[c % _NBUF],
            gsems[c % _NBUF],
        )

    def put(c):
        return pltpu.async_copy(
            rows_v.at[c % _NBUF],
            out_hbm.at[b, pl.ds(s0 + c * _R, _R)],
            osems[c % _NBUF],
        )

    gd = [gather(c) for c in range(_NBUF - 1)] + [None] * (_NCHUNK - _NBUF + 1)
    od = [None] * _NCHUNK
    for c in range(_NCHUNK):
        gd[c].wait()
        od[c] = put(c)
        nxt = c + _NBUF - 1
        if nxt < _NCHUNK:
            if nxt - _NBUF >= 0:
                od[nxt - _NBUF].wait()
            gd[nxt] = gather(nxt)
    for c in range(_NCHUNK - _NBUF, _NCHUNK):
        od[c].wait()


def kernel(test_input, emb_table):
    idx = test_input.reshape(_NW * _NCHUNK, _R)
    return _emb_lookup(emb_table, idx)


# speedup vs baseline: 1.0322x; 1.0322x over previous
import functools

import jax
import jax.numpy as jnp
from jax import lax
from jax.experimental import pallas as pl
from jax.experimental.pallas import tpu as pltpu
from jax.experimental.pallas import tpu_sc as plsc

_VOCAB = 32128
_DIM = 4096
_B = 4
_S = 512
_N = _B * _S
_NC = 2
_NS = 16
_NW = _NC * _NS
_RPW = _N // _NW
_WPB = _S // _RPW
_R = 4
_NCHUNK = _RPW // _R
_NBUF = 7

_mesh = plsc.VectorSubcoreMesh(core_axis_name="c", subcore_axis_name="s")


@functools.partial(
    pl.kernel,
    out_type=jax.ShapeDtypeStruct((_B, _S, _DIM), jnp.float32),
    mesh=_mesh,
    scratch_types=[
        pltpu.VMEM((_NCHUNK, _R), jnp.int32),
        pltpu.VMEM((_NBUF, _R, _DIM), jnp.float32),
        [pltpu.SemaphoreType.DMA] * _NBUF,
        [pltpu.SemaphoreType.DMA] * _NBUF,
    ],
)
def _emb_lookup(table_hbm, idx_hbm, out_hbm, idx_v, rows_v, gsems, osems):
    wid = lax.axis_index("s") * _NC + lax.axis_index("c")
    b = wid // _WPB
    s0 = (wid % _WPB) * _RPW
    pltpu.sync_copy(idx_hbm.at[pl.ds(wid * _NCHUNK, _NCHUNK)], idx_v)

    def gather(c):
        return pltpu.async_copy(
            table_hbm.at[idx_v.at[c]],
            rows_v.at[c % _NBUF],
            gsems[c % _NBUF],
        )

    def put(c):
        return pltpu.async_copy(
            rows_v.at[c % _NBUF],
            out_hbm.at[b, pl.ds(s0 + c * _R, _R)],
            osems[c % _NBUF],
        )

    gd = [gather(c) for c in range(_NBUF - 1)] + [None] * (_NCHUNK - _NBUF + 1)
    od = [None] * _NCHUNK
    for c in range(_NCHUNK):
        gd[c].wait()
        od[c] = put(c)
        nxt = c + _NBUF - 1
        if nxt < _NCHUNK:
            if nxt - _NBUF >= 0:
                od[nxt - _NBUF].wait()
            gd[nxt] = gather(nxt)
    for c in range(_NCHUNK - _NBUF, _NCHUNK):
        od[c].wait()


def kernel(test_input, emb_table):
    idx = test_input.reshape(_NW * _NCHUNK, _R)
    return _emb_lookup(emb_table, idx)
